# baseline (speedup 1.0000x reference)
import functools

import jax
import jax.numpy as jnp
import numpy as np
from jax import lax
from jax.experimental import pallas as pl
from jax.experimental.pallas import tpu as pltpu
from jax.experimental.pallas import tpu_sc as plsc

N_NODES = 10000
N_EDGES = 320000
N_LINKS = 65536
IN = 128
HID = 256
EMB = 128
PHID = 256
NUM_HOPS = 2
NUM_PERM = 128
HLL_P = 8
HLL_M = 1 << HLL_P
SF_DIM = (NUM_HOPS + 1) ** 2 + 2 * NUM_HOPS

_NC = 2
_NS = 16
_NW = _NC * _NS
_RPW = 320
_NPAD = _NW * _RPW
_EW = 4000
_NWIN_B = N_EDGES // _EW
_STG = _EW + 16
_CAP = N_EDGES + 16384
_K = 128

_MESH = dict(core_axis_name="c", subcore_axis_name="s")


def _wid():
    return lax.axis_index("s") * _NC + lax.axis_index("c")


def _m8(v):
    return pl.multiple_of(v, 8)


def _bucket_body(src_hbm, dst_hbm, esrc_hbm, edst_hbm, cnt_hbm, dinv_hbm,
                 dwin, swin, stg_s, stg_d, degb, cntv):
    w = _wid()
    lo = w * _RPW
    hi = lo + _RPW
    eb = w * _CAP
    dump_d = jnp.full((16,), _RPW, jnp.int32)
    dump_s = jnp.zeros((16,), jnp.int32)

    def win(i, total):
        pltpu.sync_copy(dst_hbm.at[pl.ds(i * _EW, _EW)], dwin.at[pl.ds(0, _EW)])
        pltpu.sync_copy(src_hbm.at[pl.ds(i * _EW, _EW)], swin)

        lane = lax.iota(jnp.int32, 16)

        def inner(k, st):
            sl = pl.ds(k * 16, 16)
            d16 = dwin[sl]
            s16 = swin[sl]
            m = (d16 >= lo) & (d16 < hi)
            cum = plsc.cumsum(m.astype(jnp.int32))
            pos = jnp.where(m, st + cum - 1, _STG + lane)
            plsc.store_scatter(stg_d, [pos], d16 - lo)
            plsc.store_scatter(stg_s, [pos], s16)
            pc = plsc.all_reduce_population_count(m)
            return st + pc[0]

        st = lax.fori_loop(0, _EW // 16, inner, jnp.int32(0))
        stg_d[pl.ds(st, 16)] = dump_d
        stg_s[pl.ds(st, 16)] = dump_s
        stp = jnp.bitwise_and(st + 7, jnp.int32(-8))
        pltpu.sync_copy(stg_d.at[pl.ds(0, _STG)], edst_hbm.at[pl.ds(_m8(eb + total), _STG)])
        pltpu.sync_copy(stg_s.at[pl.ds(0, _STG)], esrc_hbm.at[pl.ds(_m8(eb + total), _STG)])
        return total + stp

    total = lax.fori_loop(0, _NWIN_B, win, jnp.int32(0))

    def filldump(k, _):
        sl = pl.ds(k * 16, 16)
        stg_d[sl] = dump_d
        stg_s[sl] = dump_s
        return 0

    lax.fori_loop(0, _STG // 16, filldump, 0)
    pltpu.sync_copy(stg_d.at[pl.ds(0, _STG)], edst_hbm.at[pl.ds(_m8(eb + total), _STG)])
    pltpu.sync_copy(stg_s.at[pl.ds(0, _STG)], esrc_hbm.at[pl.ds(_m8(eb + total), _STG)])

    cntv[...] = jnp.zeros((16,), jnp.int32) + total
    pltpu.sync_copy(cntv, cnt_hbm.at[pl.ds(_m8(w * 16), 16)])

    def initdeg(j, _):
        degb[pl.ds(j * 16, 16)] = jnp.ones((16,), jnp.float32)
        return 0

    lax.fori_loop(0, 320 // 16, initdeg, 0)

    one0 = (lax.iota(jnp.int32, 16) == 0).astype(jnp.float32)

    def degwin(i, _):
        pltpu.sync_copy(edst_hbm.at[pl.ds(_m8(eb + i * _EW), _EW)], dwin.at[pl.ds(0, _EW)])

        def dedge(e, _):
            dl = dwin[pl.ds(e, 16)][0]
            degb[pl.ds(dl, 16)] = degb[pl.ds(dl, 16)] + one0
            return 0

        lax.fori_loop(0, _EW, dedge, 0)
        return 0

    nw2 = (total + _EW - 1) // _EW
    lax.fori_loop(0, nw2, degwin, 0)

    def dj(j, _):
        sl = pl.ds(j * 16, 16)
        d = degb[sl]
        i = jnp.int32(0x5F3759DF) - (plsc.bitcast(d, jnp.int32) >> 1)
        y = plsc.bitcast(i, jnp.float32)
        for _r in range(3):
            y = y * (1.5 - 0.5 * d * y * y)
        degb[sl] = y
        return 0

    lax.fori_loop(0, 320 // 16, dj, 0)
    pltpu.sync_copy(degb.at[pl.ds(0, _RPW)], dinv_hbm.at[pl.ds(_m8(w * _RPW), _RPW)])


_bucket = functools.partial(
    pl.kernel,
    out_type=[
        jax.ShapeDtypeStruct((_NW * _CAP,), jnp.int32),
        jax.ShapeDtypeStruct((_NW * _CAP,), jnp.int32),
        jax.ShapeDtypeStruct((_NW * 16,), jnp.int32),
        jax.ShapeDtypeStruct((_NW * _RPW,), jnp.float32),
    ],
    mesh=plsc.VectorSubcoreMesh(**_MESH),
    compiler_params=pltpu.CompilerParams(needs_layout_passes=False),
    scratch_types=[
        pltpu.VMEM((_EW + 16,), jnp.int32),
        pltpu.VMEM((_EW,), jnp.int32),
        pltpu.VMEM((_STG + 16,), jnp.int32),
        pltpu.VMEM((_STG + 16,), jnp.int32),
        pltpu.VMEM((336,), jnp.float32),
        pltpu.VMEM((16,), jnp.int32),
    ],
)(_bucket_body)


def _make_prop(W, op):
    def body(tab_hbm, esrc_hbm, edst_hbm, cnt_hbm, out_hbm,
             acc, rows, idxv, dlv, cv, sem):
        w = _wid()
        lo = w * _RPW
        eb = w * _CAP
        pltpu.sync_copy(cnt_hbm, cv)
        cnt = cv[pl.ds(_m8(w * 16), 16)][0]
        pltpu.sync_copy(tab_hbm.at[pl.ds(_m8(lo), _RPW)], acc.at[pl.ds(0, _RPW)])
        nwin = (cnt + (_K - 1)) // _K

        def win(i, _):
            base = eb + i * _K
            pltpu.sync_copy(esrc_hbm.at[pl.ds(_m8(base), _K)], idxv)
            pltpu.sync_copy(edst_hbm.at[pl.ds(_m8(base), _K)], dlv.at[pl.ds(0, _K)])
            pltpu.async_copy(tab_hbm.at[idxv], rows, sem).wait()

            def edge(e, _):
                dl = dlv[pl.ds(e, 16)][0]
                for j in range(W // 16):
                    sl = pl.ds(j * 16, 16)
                    acc[dl, sl] = op(acc[dl, sl], rows[e, sl])
                return 0

            lax.fori_loop(0, _K, edge, 0, unroll=4)
            return 0

        lax.fori_loop(0, nwin, win, 0)
        pltpu.sync_copy(acc.at[pl.ds(0, _RPW)], out_hbm.at[w])

    return pl.kernel(
        body,
        out_type=jax.ShapeDtypeStruct((_NW, _RPW, W), jnp.int32),
        mesh=plsc.VectorSubcoreMesh(**_MESH),
        compiler_params=pltpu.CompilerParams(needs_layout_passes=False),
        scratch_types=[
            pltpu.VMEM((_RPW + 1, W), jnp.int32),
            pltpu.VMEM((_K, W), jnp.int32),
            pltpu.VMEM((_K,), jnp.int32),
            pltpu.VMEM((_K + 16,), jnp.int32),
            pltpu.VMEM((_NW * 16,), jnp.int32),
            pltpu.SemaphoreType.DMA,
        ],
    )


def _max16(a, b):
    a16 = plsc.bitcast(a, jnp.int16)
    b16 = plsc.bitcast(b, jnp.int16)
    return plsc.bitcast(jnp.maximum(a16, b16), jnp.int32)


_prop_min_128 = _make_prop(128, jnp.minimum)
_prop_max_128p = _make_prop(128, _max16)


def _make_gcn(W):
    def body(hs_hbm, esrc_hbm, edst_hbm, cnt_hbm, dinv_hbm, b_hbm, out_hbm,
             acc, rows, idxv, dlv, cv, dv, bv, sem):
        w = _wid()
        lo = w * _RPW
        eb = w * _CAP
        pltpu.sync_copy(cnt_hbm, cv)
        cnt = cv[pl.ds(_m8(w * 16), 16)][0]
        pltpu.sync_copy(hs_hbm.at[pl.ds(_m8(lo), _RPW)], acc.at[pl.ds(0, _RPW)])
        nwin = (cnt + (_K - 1)) // _K

        def win(i, _):
            base = eb + i * _K
            pltpu.sync_copy(esrc_hbm.at[pl.ds(_m8(base), _K)], idxv)
            pltpu.sync_copy(edst_hbm.at[pl.ds(_m8(base), _K)], dlv.at[pl.ds(0, _K)])
            pltpu.async_copy(hs_hbm.at[idxv], rows, sem).wait()

            def edge(e, _):
                dl = dlv[pl.ds(e, 16)][0]
                for j in range(W // 16):
                    sl = pl.ds(j * 16, 16)
                    acc[dl, sl] = acc[dl, sl] + rows[e, sl]
                return 0

            lax.fori_loop(0, _K, edge, 0, unroll=4)
            return 0

        lax.fori_loop(0, nwin, win, 0)

        pltpu.sync_copy(dinv_hbm.at[pl.ds(_m8(w * _RPW), _RPW)], dv.at[pl.ds(0, _RPW)])
        pltpu.sync_copy(b_hbm, bv)

        def row(r, _):
            s = dv[pl.ds(r, 16)][0]
            for j in range(W // 16):
                sl = pl.ds(j * 16, 16)
                acc[r, sl] = acc[r, sl] * s + bv[sl]
            return 0

        lax.fori_loop(0, _RPW, row, 0)
        pltpu.sync_copy(acc.at[pl.ds(0, _RPW)], out_hbm.at[w])

    return pl.kernel(
        body,
        out_type=jax.ShapeDtypeStruct((_NW, _RPW, W), jnp.float32),
        mesh=plsc.VectorSubcoreMesh(**_MESH),
        compiler_params=pltpu.CompilerParams(needs_layout_passes=False),
        scratch_types=[
            pltpu.VMEM((_RPW + 1, W), jnp.float32),
            pltpu.VMEM((_K, W), jnp.float32),
            pltpu.VMEM((_K,), jnp.int32),
            pltpu.VMEM((_K + 16,), jnp.int32),
            pltpu.VMEM((_NW * 16,), jnp.int32),
            pltpu.VMEM((336,), jnp.float32),
            pltpu.VMEM((W,), jnp.float32),
            pltpu.SemaphoreType.DMA,
        ],
    )


_gcn_256 = _make_gcn(256)
_gcn_128 = _make_gcn(128)


def _mm_scale_body(relu_in, x_ref, w_ref, d_ref, o_ref):
    xv = x_ref[...]
    if relu_in:
        xv = jnp.maximum(xv, 0.0)
    o_ref[...] = (
        jnp.dot(xv, w_ref[...], preferred_element_type=jnp.float32) * d_ref[...]
    )


def _mm_scale(x, W, dinv_col, relu_in):
    M, Kd = x.shape
    N = W.shape[1]
    BM = 2560
    return pl.pallas_call(
        functools.partial(_mm_scale_body, relu_in),
        grid=(M // BM,),
        in_specs=[
            pl.BlockSpec((BM, Kd), lambda i: (i, 0)),
            pl.BlockSpec((Kd, N), lambda i: (0, 0)),
            pl.BlockSpec((BM, 1), lambda i: (i, 0)),
        ],
        out_specs=pl.BlockSpec((BM, N), lambda i: (i, 0)),
        out_shape=jax.ShapeDtypeStruct((M, N), jnp.float32),
    )(x, W, dinv_col)


def _mlp_body(inp_ref, wp1_ref, bp1_ref, wp2_ref, bp2_ref, out_ref):
    h = jnp.dot(inp_ref[...], wp1_ref[...], preferred_element_type=jnp.float32)
    h = jnp.maximum(h + bp1_ref[...], 0.0)
    out_ref[...] = (
        jnp.dot(h, wp2_ref[...], preferred_element_type=jnp.float32) + bp2_ref[...]
    )


def _mlp(inp, Wp1, bp1, Wp2, bp2):
    B, Kd = inp.shape
    BL = 2048
    out = pl.pallas_call(
        _mlp_body,
        grid=(B // BL,),
        in_specs=[
            pl.BlockSpec((BL, Kd), lambda i: (i, 0)),
            pl.BlockSpec((Kd, PHID), lambda i: (0, 0)),
            pl.BlockSpec((1, PHID), lambda i: (0, 0)),
            pl.BlockSpec((PHID, 1), lambda i: (0, 0)),
            pl.BlockSpec((1, 1), lambda i: (0, 0)),
        ],
        out_specs=pl.BlockSpec((BL, 1), lambda i: (i, 0)),
        out_shape=jax.ShapeDtypeStruct((B, 1), jnp.float32),
    )(inp, Wp1, bp1.reshape(1, PHID), Wp2, bp2.reshape(1, 1))
    return out.reshape(-1)


def _init_minhash_np(n):
    rng = np.random.default_rng(0)
    prime = (1 << 31) - 1
    a = rng.integers(1, prime, size=NUM_PERM, dtype=np.int64)
    b = rng.integers(0, prime, size=NUM_PERM, dtype=np.int64)
    v = np.arange(n, dtype=np.int64)[:, None]
    mh = (a[None, :] * v + b[None, :]) % prime
    return jnp.asarray(mh.astype(np.int32))


def _init_hll_np(n):
    v = np.arange(n, dtype=np.uint64)
    h = ((v * np.uint64(0x9E3779B97F4A7C15)) & np.uint64(0xFFFFFFFF)).astype(np.int64)
    idx = (h & (HLL_M - 1)).astype(np.int64)
    bits = 32 - HLL_P
    w = (h >> HLL_P) & ((1 << bits) - 1)
    msb = np.floor(np.log2(np.maximum(w, 1))).astype(np.int64)
    rho = np.where(w == 0, bits + 1, bits - msb)
    regs = np.zeros((n, HLL_M), dtype=np.int16)
    regs[np.arange(n), idx] = rho.astype(np.int16)
    return jnp.asarray(regs.view(np.int32))


def _hll_card(regs):
    m = float(HLL_M)
    alpha = 0.7213 / (1.0 + 1.079 / m)
    z = jnp.sum(jnp.exp2(-regs.astype(jnp.float32)), axis=-1)
    return alpha * m * m / z


def kernel(x, edge_index, edge_label_index, W1, b1, W2, b2, Wp1, bp1, Wp2, bp2):
    src = edge_index[0]
    dst = edge_index[1]

    esrc, edst, cnts, dinv_flat = _bucket(src, dst)
    dinv_col = dinv_flat.reshape(_NPAD, 1)

    xp = jnp.pad(x, ((0, _NPAD - N_NODES), (0, 0)))
    hs1 = _mm_scale(xp, W1, dinv_col, relu_in=False)
    z1 = _gcn_256(hs1, esrc, edst, cnts, dinv_flat, b1).reshape(_NPAD, HID)
    hs2 = _mm_scale(z1, W2, dinv_col, relu_in=True)
    z = _gcn_128(hs2, esrc, edst, cnts, dinv_flat, b2).reshape(_NPAD, EMB)[:N_NODES]

    mh0 = _init_minhash_np(_NPAD)
    mh1 = _prop_min_128(mh0, esrc, edst, cnts).reshape(_NPAD, NUM_PERM)
    mh2 = _prop_min_128(mh1, esrc, edst, cnts).reshape(_NPAD, NUM_PERM)
    hll0p = _init_hll_np(_NPAD)
    hll1p = _prop_max_128p(hll0p, esrc, edst, cnts).reshape(_NPAD, HLL_M // 2)
    hll2p = _prop_max_128p(hll1p, esrc, edst, cnts).reshape(_NPAD, HLL_M // 2)

    def _unpack(hp):
        h16 = lax.bitcast_convert_type(hp, jnp.int16)
        return h16.reshape(_NPAD, HLL_M)[:N_NODES].astype(jnp.int32)

    mh = [mh0[:N_NODES], mh1[:N_NODES], mh2[:N_NODES]]
    hll = [_unpack(hll0p), _unpack(hll1p), _unpack(hll2p)]

    ls = edge_label_index[0]
    ld = edge_label_index[1]
    feats = []
    for i in range(NUM_HOPS + 1):
        for j in range(NUM_HOPS + 1):
            jac = jnp.mean((mh[i][ls] == mh[j][ld]).astype(jnp.float32), axis=-1)
            union = _hll_card(jnp.maximum(hll[i][ls], hll[j][ld]).astype(jnp.int32))
            feats.append(jac * union)
    for i in range(1, NUM_HOPS + 1):
        feats.append(_hll_card(hll[i][ls].astype(jnp.int32)))
    for i in range(1, NUM_HOPS + 1):
        feats.append(_hll_card(hll[i][ld].astype(jnp.int32)))
    sf = jnp.stack(feats, axis=-1)
    zs = z[ls]
    zd = z[ld]
    pair = jnp.concatenate([zs, zd, zs * zd, jnp.abs(zs - zd)], axis=-1)
    inp = jnp.concatenate([pair, sf], axis=-1)
    return _mlp(inp, Wp1, bp1, Wp2, bp2)


# speedup vs baseline: 1.0697x; 1.0697x over previous
import functools

import jax
import jax.numpy as jnp
import numpy as np
from jax import lax
from jax.experimental import pallas as pl
from jax.experimental.pallas import tpu as pltpu
from jax.experimental.pallas import tpu_sc as plsc

N_NODES = 10000
N_EDGES = 320000
N_LINKS = 65536
IN = 128
HID = 256
EMB = 128
PHID = 256
NUM_HOPS = 2
NUM_PERM = 128
HLL_P = 8
HLL_M = 1 << HLL_P
SF_DIM = (NUM_HOPS + 1) ** 2 + 2 * NUM_HOPS

_NC = 2
_NS = 16
_NW = _NC * _NS
_RPW = 320
_NPAD = _NW * _RPW
_EW = 4000
_NWIN_B = N_EDGES // _EW
_STG = _EW + 16
_CAP = N_EDGES + 16384
_K = 128

_MESH = dict(core_axis_name="c", subcore_axis_name="s")


def _wid():
    return lax.axis_index("s") * _NC + lax.axis_index("c")


def _m8(v):
    return pl.multiple_of(v, 8)


def _bucket_body(src_hbm, dst_hbm, esrc_hbm, edst_hbm, cnt_hbm, dinv_hbm,
                 dwin, swin, stg_s, stg_d, degb, cntv):
    w = _wid()
    lo = w * _RPW
    hi = lo + _RPW
    eb = w * _CAP
    dump_d = jnp.full((16,), _RPW, jnp.int32)
    dump_s = jnp.zeros((16,), jnp.int32)

    def win(i, total):
        pltpu.sync_copy(dst_hbm.at[pl.ds(i * _EW, _EW)], dwin.at[pl.ds(0, _EW)])
        pltpu.sync_copy(src_hbm.at[pl.ds(i * _EW, _EW)], swin)

        lane = lax.iota(jnp.int32, 16)

        def inner(k, st):
            sl = pl.ds(k * 16, 16)
            d16 = dwin[sl]
            s16 = swin[sl]
            m = (d16 >= lo) & (d16 < hi)
            cum = plsc.cumsum(m.astype(jnp.int32))
            pos = jnp.where(m, st + cum - 1, _STG + lane)
            plsc.store_scatter(stg_d, [pos], d16 - lo)
            plsc.store_scatter(stg_s, [pos], s16)
            pc = plsc.all_reduce_population_count(m)
            return st + pc[0]

        st = lax.fori_loop(0, _EW // 16, inner, jnp.int32(0))
        stg_d[pl.ds(st, 16)] = dump_d
        stg_s[pl.ds(st, 16)] = dump_s
        stp = jnp.bitwise_and(st + 7, jnp.int32(-8))
        pltpu.sync_copy(stg_d.at[pl.ds(0, _STG)], edst_hbm.at[pl.ds(_m8(eb + total), _STG)])
        pltpu.sync_copy(stg_s.at[pl.ds(0, _STG)], esrc_hbm.at[pl.ds(_m8(eb + total), _STG)])
        return total + stp

    total = lax.fori_loop(0, _NWIN_B, win, jnp.int32(0))

    def filldump(k, _):
        sl = pl.ds(k * 16, 16)
        stg_d[sl] = dump_d
        stg_s[sl] = dump_s
        return 0

    lax.fori_loop(0, _STG // 16, filldump, 0)
    pltpu.sync_copy(stg_d.at[pl.ds(0, _STG)], edst_hbm.at[pl.ds(_m8(eb + total), _STG)])
    pltpu.sync_copy(stg_s.at[pl.ds(0, _STG)], esrc_hbm.at[pl.ds(_m8(eb + total), _STG)])

    cntv[...] = jnp.zeros((16,), jnp.int32) + total
    pltpu.sync_copy(cntv, cnt_hbm.at[pl.ds(_m8(w * 16), 16)])

    def initdeg(j, _):
        degb[pl.ds(j * 16, 16)] = jnp.ones((16,), jnp.float32)
        return 0

    lax.fori_loop(0, 320 // 16, initdeg, 0)

    one0 = (lax.iota(jnp.int32, 16) == 0).astype(jnp.float32)

    def degwin(i, _):
        pltpu.sync_copy(edst_hbm.at[pl.ds(_m8(eb + i * _EW), _EW)], dwin.at[pl.ds(0, _EW)])

        def dedge(e, _):
            dl = dwin[pl.ds(e, 16)][0]
            degb[pl.ds(dl, 16)] = degb[pl.ds(dl, 16)] + one0
            return 0

        lax.fori_loop(0, _EW, dedge, 0)
        return 0

    nw2 = (total + _EW - 1) // _EW
    lax.fori_loop(0, nw2, degwin, 0)

    def dj(j, _):
        sl = pl.ds(j * 16, 16)
        d = degb[sl]
        i = jnp.int32(0x5F3759DF) - (plsc.bitcast(d, jnp.int32) >> 1)
        y = plsc.bitcast(i, jnp.float32)
        for _r in range(3):
            y = y * (1.5 - 0.5 * d * y * y)
        degb[sl] = y
        return 0

    lax.fori_loop(0, 320 // 16, dj, 0)
    pltpu.sync_copy(degb.at[pl.ds(0, _RPW)], dinv_hbm.at[pl.ds(_m8(w * _RPW), _RPW)])


_bucket = functools.partial(
    pl.kernel,
    out_type=[
        jax.ShapeDtypeStruct((_NW * _CAP,), jnp.int32),
        jax.ShapeDtypeStruct((_NW * _CAP,), jnp.int32),
        jax.ShapeDtypeStruct((_NW * 16,), jnp.int32),
        jax.ShapeDtypeStruct((_NW * _RPW,), jnp.float32),
    ],
    mesh=plsc.VectorSubcoreMesh(**_MESH),
    compiler_params=pltpu.CompilerParams(needs_layout_passes=False),
    scratch_types=[
        pltpu.VMEM((_EW + 16,), jnp.int32),
        pltpu.VMEM((_EW,), jnp.int32),
        pltpu.VMEM((_STG + 16,), jnp.int32),
        pltpu.VMEM((_STG + 16,), jnp.int32),
        pltpu.VMEM((336,), jnp.float32),
        pltpu.VMEM((16,), jnp.int32),
    ],
)(_bucket_body)


def _make_prop(W, op):
    def body(tab_hbm, esrc_hbm, edst_hbm, cnt_hbm, out_hbm,
             acc, rows, idxv, dlv, cv, sem):
        w = _wid()
        lo = w * _RPW
        eb = w * _CAP
        pltpu.sync_copy(cnt_hbm, cv)
        cnt = cv[pl.ds(_m8(w * 16), 16)][0]
        pltpu.sync_copy(tab_hbm.at[pl.ds(_m8(lo), _RPW)], acc.at[pl.ds(0, _RPW)])
        nwin = (cnt + (_K - 1)) // _K

        def win(i, _):
            base = eb + i * _K
            pltpu.sync_copy(esrc_hbm.at[pl.ds(_m8(base), _K)], idxv)
            pltpu.sync_copy(edst_hbm.at[pl.ds(_m8(base), _K)], dlv.at[pl.ds(0, _K)])
            pltpu.async_copy(tab_hbm.at[idxv], rows, sem).wait()

            def edge(e, _):
                dl = dlv[pl.ds(e, 16)][0]
                for j in range(W // 16):
                    sl = pl.ds(j * 16, 16)
                    acc[dl, sl] = op(acc[dl, sl], rows[e, sl])
                return 0

            lax.fori_loop(0, _K, edge, 0, unroll=8)
            return 0

        lax.fori_loop(0, nwin, win, 0)
        pltpu.sync_copy(acc.at[pl.ds(0, _RPW)], out_hbm.at[w])

    return pl.kernel(
        body,
        out_type=jax.ShapeDtypeStruct((_NW, _RPW, W), jnp.int32),
        mesh=plsc.VectorSubcoreMesh(**_MESH),
        compiler_params=pltpu.CompilerParams(needs_layout_passes=False),
        scratch_types=[
            pltpu.VMEM((_RPW + 1, W), jnp.int32),
            pltpu.VMEM((_K, W), jnp.int32),
            pltpu.VMEM((_K,), jnp.int32),
            pltpu.VMEM((_K + 16,), jnp.int32),
            pltpu.VMEM((_NW * 16,), jnp.int32),
            pltpu.SemaphoreType.DMA,
        ],
    )


_prop_min_128 = _make_prop(128, jnp.minimum)
_prop_max_256 = _make_prop(256, jnp.maximum)


def _make_gcn(W):
    def body(hs_hbm, esrc_hbm, edst_hbm, cnt_hbm, dinv_hbm, b_hbm, out_hbm,
             acc, rows, idxv, dlv, cv, dv, bv, sem):
        w = _wid()
        lo = w * _RPW
        eb = w * _CAP
        pltpu.sync_copy(cnt_hbm, cv)
        cnt = cv[pl.ds(_m8(w * 16), 16)][0]
        pltpu.sync_copy(hs_hbm.at[pl.ds(_m8(lo), _RPW)], acc.at[pl.ds(0, _RPW)])
        nwin = (cnt + (_K - 1)) // _K

        def win(i, _):
            base = eb + i * _K
            pltpu.sync_copy(esrc_hbm.at[pl.ds(_m8(base), _K)], idxv)
            pltpu.sync_copy(edst_hbm.at[pl.ds(_m8(base), _K)], dlv.at[pl.ds(0, _K)])
            pltpu.async_copy(hs_hbm.at[idxv], rows, sem).wait()

            def edge(e, _):
                dl = dlv[pl.ds(e, 16)][0]
                for j in range(W // 16):
                    sl = pl.ds(j * 16, 16)
                    acc[dl, sl] = acc[dl, sl] + rows[e, sl]
                return 0

            lax.fori_loop(0, _K, edge, 0, unroll=8)
            return 0

        lax.fori_loop(0, nwin, win, 0)

        pltpu.sync_copy(dinv_hbm.at[pl.ds(_m8(w * _RPW), _RPW)], dv.at[pl.ds(0, _RPW)])
        pltpu.sync_copy(b_hbm, bv)

        def row(r, _):
            s = dv[pl.ds(r, 16)][0]
            for j in range(W // 16):
                sl = pl.ds(j * 16, 16)
                acc[r, sl] = acc[r, sl] * s + bv[sl]
            return 0

        lax.fori_loop(0, _RPW, row, 0)
        pltpu.sync_copy(acc.at[pl.ds(0, _RPW)], out_hbm.at[w])

    return pl.kernel(
        body,
        out_type=jax.ShapeDtypeStruct((_NW, _RPW, W), jnp.float32),
        mesh=plsc.VectorSubcoreMesh(**_MESH),
        compiler_params=pltpu.CompilerParams(needs_layout_passes=False),
        scratch_types=[
            pltpu.VMEM((_RPW + 1, W), jnp.float32),
            pltpu.VMEM((_K, W), jnp.float32),
            pltpu.VMEM((_K,), jnp.int32),
            pltpu.VMEM((_K + 16,), jnp.int32),
            pltpu.VMEM((_NW * 16,), jnp.int32),
            pltpu.VMEM((336,), jnp.float32),
            pltpu.VMEM((W,), jnp.float32),
            pltpu.SemaphoreType.DMA,
        ],
    )


_gcn_256 = _make_gcn(256)
_gcn_128 = _make_gcn(128)


def _mm_scale_body(relu_in, x_ref, w_ref, d_ref, o_ref):
    xv = x_ref[...]
    if relu_in:
        xv = jnp.maximum(xv, 0.0)
    o_ref[...] = (
        jnp.dot(xv, w_ref[...], preferred_element_type=jnp.float32) * d_ref[...]
    )


def _mm_scale(x, W, dinv_col, relu_in):
    M, Kd = x.shape
    N = W.shape[1]
    BM = 2560
    return pl.pallas_call(
        functools.partial(_mm_scale_body, relu_in),
        grid=(M // BM,),
        in_specs=[
            pl.BlockSpec((BM, Kd), lambda i: (i, 0)),
            pl.BlockSpec((Kd, N), lambda i: (0, 0)),
            pl.BlockSpec((BM, 1), lambda i: (i, 0)),
        ],
        out_specs=pl.BlockSpec((BM, N), lambda i: (i, 0)),
        out_shape=jax.ShapeDtypeStruct((M, N), jnp.float32),
    )(x, W, dinv_col)


def _mlp_body(inp_ref, wp1_ref, bp1_ref, wp2_ref, bp2_ref, out_ref):
    h = jnp.dot(inp_ref[...], wp1_ref[...], preferred_element_type=jnp.float32)
    h = jnp.maximum(h + bp1_ref[...], 0.0)
    out_ref[...] = (
        jnp.dot(h, wp2_ref[...], preferred_element_type=jnp.float32) + bp2_ref[...]
    )


def _mlp(inp, Wp1, bp1, Wp2, bp2):
    B, Kd = inp.shape
    BL = 2048
    out = pl.pallas_call(
        _mlp_body,
        grid=(B // BL,),
        in_specs=[
            pl.BlockSpec((BL, Kd), lambda i: (i, 0)),
            pl.BlockSpec((Kd, PHID), lambda i: (0, 0)),
            pl.BlockSpec((1, PHID), lambda i: (0, 0)),
            pl.BlockSpec((PHID, 1), lambda i: (0, 0)),
            pl.BlockSpec((1, 1), lambda i: (0, 0)),
        ],
        out_specs=pl.BlockSpec((BL, 1), lambda i: (i, 0)),
        out_shape=jax.ShapeDtypeStruct((B, 1), jnp.float32),
    )(inp, Wp1, bp1.reshape(1, PHID), Wp2, bp2.reshape(1, 1))
    return out.reshape(-1)


def _init_minhash_np(n):
    rng = np.random.default_rng(0)
    prime = (1 << 31) - 1
    a = rng.integers(1, prime, size=NUM_PERM, dtype=np.int64)
    b = rng.integers(0, prime, size=NUM_PERM, dtype=np.int64)
    v = np.arange(n, dtype=np.int64)[:, None]
    mh = (a[None, :] * v + b[None, :]) % prime
    return jnp.asarray(mh.astype(np.int32))


def _init_hll_np(n):
    v = np.arange(n, dtype=np.uint64)
    h = ((v * np.uint64(0x9E3779B97F4A7C15)) & np.uint64(0xFFFFFFFF)).astype(np.int64)
    idx = (h & (HLL_M - 1)).astype(np.int64)
    bits = 32 - HLL_P
    w = (h >> HLL_P) & ((1 << bits) - 1)
    msb = np.floor(np.log2(np.maximum(w, 1))).astype(np.int64)
    rho = np.where(w == 0, bits + 1, bits - msb)
    regs = np.zeros((n, HLL_M), dtype=np.int32)
    regs[np.arange(n), idx] = rho.astype(np.int32)
    return jnp.asarray(regs)


def _hll_card(regs):
    m = float(HLL_M)
    alpha = 0.7213 / (1.0 + 1.079 / m)
    z = jnp.sum(jnp.exp2(-regs.astype(jnp.float32)), axis=-1)
    return alpha * m * m / z


def kernel(x, edge_index, edge_label_index, W1, b1, W2, b2, Wp1, bp1, Wp2, bp2):
    src = edge_index[0]
    dst = edge_index[1]

    esrc, edst, cnts, dinv_flat = _bucket(src, dst)
    dinv_col = dinv_flat.reshape(_NPAD, 1)

    xp = jnp.pad(x, ((0, _NPAD - N_NODES), (0, 0)))
    hs1 = _mm_scale(xp, W1, dinv_col, relu_in=False)
    z1 = _gcn_256(hs1, esrc, edst, cnts, dinv_flat, b1).reshape(_NPAD, HID)
    hs2 = _mm_scale(z1, W2, dinv_col, relu_in=True)
    z = _gcn_128(hs2, esrc, edst, cnts, dinv_flat, b2).reshape(_NPAD, EMB)[:N_NODES]

    mh0 = _init_minhash_np(_NPAD)
    mh1 = _prop_min_128(mh0, esrc, edst, cnts).reshape(_NPAD, NUM_PERM)
    mh2 = _prop_min_128(mh1, esrc, edst, cnts).reshape(_NPAD, NUM_PERM)
    hll0 = _init_hll_np(_NPAD)
    hll1 = _prop_max_256(hll0, esrc, edst, cnts).reshape(_NPAD, HLL_M)
    hll2 = _prop_max_256(hll1, esrc, edst, cnts).reshape(_NPAD, HLL_M)

    mh = [mh0[:N_NODES], mh1[:N_NODES], mh2[:N_NODES]]
    hll = [hll0[:N_NODES], hll1[:N_NODES], hll2[:N_NODES]]

    ls = edge_label_index[0]
    ld = edge_label_index[1]
    feats = []
    for i in range(NUM_HOPS + 1):
        for j in range(NUM_HOPS + 1):
            jac = jnp.mean((mh[i][ls] == mh[j][ld]).astype(jnp.float32), axis=-1)
            union = _hll_card(jnp.maximum(hll[i][ls], hll[j][ld]))
            feats.append(jac * union)
    for i in range(1, NUM_HOPS + 1):
        feats.append(_hll_card(hll[i][ls]))
    for i in range(1, NUM_HOPS + 1):
        feats.append(_hll_card(hll[i][ld]))
    sf = jnp.stack(feats, axis=-1)
    zs = z[ls]
    zd = z[ld]
    pair = jnp.concatenate([zs, zd, zs * zd, jnp.abs(zs - zd)], axis=-1)
    inp = jnp.concatenate([pair, sf], axis=-1)
    return _mlp(inp, Wp1, bp1, Wp2, bp2)
